# baseline (speedup 1.0000x reference)
import jax
import jax.numpy as jnp
from jax import lax
from jax.experimental import pallas as pl
from jax.experimental.pallas import tpu as pltpu
from jax.experimental.pallas import tpu_sc as plsc

_BATCH = 16
_SEQ = 2048
_OUT = 256
_T = _BATCH * _SEQ * _OUT
_NNZ = 1000000

_NC = 2
_NS = 16
_NW = _NC * _NS

_SUB = 32768
_NSUB = _T // _SUB
_SUB_PER_W = _NSUB // _NW
_E = 4096


def _sc_body(flat_hbm, val_hbm, out_hbm, gbuf, gsem, stage0, stage1, osem0,
             osem1, fbuf0, fbuf1, vbuf0, vbuf1, fsem, vsem):
    stages_l = [stage0, stage1]
    osems_l = [osem0, osem1]
    fbufs_l = [fbuf0, fbuf1]
    vbufs_l = [vbuf0, vbuf1]
    cid = lax.axis_index("c")
    sid = lax.axis_index("s")
    wid = sid * _NC + cid
    c0 = wid * _SUB_PER_W

    lanes = lax.iota(jnp.int32, 16)
    q = (c0 + jnp.minimum(lanes, _SUB_PER_W)) * _SUB
    zero16 = jnp.zeros((16,), jnp.float32)

    blo = jnp.zeros((16,), jnp.int32)
    bhi = jnp.full((16,), _NNZ, jnp.int32)
    for r in range(14):
        upd = blo < bhi
        mid = jnp.minimum((blo + bhi) >> 1, _NNZ - 1)
        pltpu.async_copy(flat_hbm.at[mid], gbuf, gsem).wait()
        lt = gbuf[...] < q
        blo = jnp.where(jnp.logical_and(upd, lt), mid + 1, blo)
        bhi = jnp.where(jnp.logical_and(upd, jnp.logical_not(lt)), mid, bhi)
    out_descs = [None, None]

    for k in range(_SUB_PER_W):
        c = c0 + k
        lo = c * _SUB
        hi = lo + _SUB
        s_lo = blo[k]
        s_hi = bhi[k + 1]
        buf = k % 2
        stage = stages_l[buf]
        fbuf = fbufs_l[buf]
        vbuf = vbufs_l[buf]

        a = (s_lo // 8) * 8
        n = s_hi - a
        nch = (n + _E - 1) // _E

        def _load(j, which):
            off = jnp.minimum(a + j * _E, _NNZ - _E)
            off = pl.multiple_of((off // 8) * 8, 8)
            if which == 0:
                return pltpu.async_copy(
                    flat_hbm.at[pl.ds(off, _E)], fbuf, fsem)
            return pltpu.async_copy(
                val_hbm.at[pl.ds(off, _E)], vbuf, vsem)

        d_f = _load(0, 0)
        d_v = _load(0, 1)

        if out_descs[buf] is not None:
            out_descs[buf].wait()

        def zbody(i, carry):
            for u in range(16):
                stage[i, pl.ds(u * 16, 16)] = zero16
            return carry

        lax.fori_loop(0, _SUB // 256, zbody, 0)

        def gloop(j):
            off = jnp.minimum(a + j * _E, _NNZ - _E)
            off = (off // 8) * 8
            nit = jnp.clip(s_hi - off, 0, _E)
            nit = (nit + 127) // 128

            def gbody(g, gc):
                for u in range(8):
                    sl = pl.ds((g * 8 + u) * 16, 16)
                    fv = fbuf[sl]
                    vv = vbuf[sl]
                    m = jnp.logical_and(fv >= lo, fv < hi)
                    d = fv - lo
                    plsc.store_scatter(
                        stage, [d >> 8, d & 255], vv, mask=m)
                return gc

            lax.fori_loop(0, nit, gbody, 0)

        d_f.wait()
        d_v.wait()
        gloop(0)

        def ebody(j, carry):
            d_f = _load(j, 0)
            d_v = _load(j, 1)
            d_f.wait()
            d_v.wait()
            gloop(j)
            return carry

        lax.fori_loop(1, nch, ebody, 0)

        out_descs[buf] = pltpu.async_copy(
            stage,
            out_hbm.at[c // 16,
                       pl.ds(pl.multiple_of((c % 16) * 128, 8), 128), :],
            osems_l[buf])

    out_descs[0].wait()
    out_descs[1].wait()


@jax.jit
def _sc_scatter(flat_p, val_p):
    mesh = plsc.VectorSubcoreMesh(
        core_axis_name="c", subcore_axis_name="s", num_cores=_NC,
        num_subcores=_NS)
    return pl.kernel(
        _sc_body,
        out_type=jax.ShapeDtypeStruct((_BATCH, _SEQ, _OUT), jnp.float32),
        mesh=mesh,
        compiler_params=pltpu.CompilerParams(needs_layout_passes=False),
        scratch_types=[
            pltpu.VMEM((16,), jnp.int32),
            pltpu.SemaphoreType.DMA,
            pltpu.VMEM((128, 256), jnp.float32),
            pltpu.VMEM((128, 256), jnp.float32),
            pltpu.SemaphoreType.DMA,
            pltpu.SemaphoreType.DMA,
            pltpu.VMEM((_E,), jnp.int32),
            pltpu.VMEM((_E,), jnp.int32),
            pltpu.VMEM((_E,), jnp.float32),
            pltpu.VMEM((_E,), jnp.float32),
            pltpu.SemaphoreType.DMA,
            pltpu.SemaphoreType.DMA,
        ],
    )(flat_p, val_p)


def kernel(indices, values):
    idx = indices.astype(jnp.int32)
    flat = idx[:, 0] * (_SEQ * _OUT) + idx[:, 1] * _OUT + idx[:, 2]
    return _sc_scatter(flat, values)
